# baseline (speedup 1.0000x reference)
import jax
import jax.numpy as jnp
from jax.experimental import pallas as pl


def _add_body(x_ref, emb_ref, o_ref):
    o_ref[...] = x_ref[...] + emb_ref[...][None]


def kernel(x, emb):
    B, S, D = x.shape
    BS = 256
    grid = (S // BS, B)
    return pl.pallas_call(
        _add_body,
        grid=grid,
        in_specs=[
            pl.BlockSpec((1, BS, D), lambda i, j: (j, i, 0)),
            pl.BlockSpec((BS, D), lambda i, j: (i, 0)),
        ],
        out_specs=pl.BlockSpec((1, BS, D), lambda i, j: (j, i, 0)),
        out_shape=jax.ShapeDtypeStruct((B, S, D), x.dtype),
    )(x, emb)


# speedup vs baseline: 1.4340x; 1.4340x over previous
import jax
import jax.numpy as jnp
from jax.experimental import pallas as pl


def _add_body(x_ref, emb_ref, o_ref):
    o_ref[...] = x_ref[...] + emb_ref[...][None]


def kernel(x, emb):
    B, S, D = x.shape
    BS = 1024
    grid = (S // BS, B)
    return pl.pallas_call(
        _add_body,
        grid=grid,
        in_specs=[
            pl.BlockSpec((1, BS, D), lambda i, j: (j, i, 0)),
            pl.BlockSpec((BS, D), lambda i, j: (i, 0)),
        ],
        out_specs=pl.BlockSpec((1, BS, D), lambda i, j: (j, i, 0)),
        out_shape=jax.ShapeDtypeStruct((B, S, D), x.dtype),
    )(x, emb)


# speedup vs baseline: 1.5478x; 1.0794x over previous
import jax
import jax.numpy as jnp
from jax.experimental import pallas as pl


def _add_body(x_ref, emb_ref, o_ref):
    o_ref[...] = x_ref[...] + emb_ref[...][None]


def kernel(x, emb):
    B, S, D = x.shape
    BS = 2048
    grid = (S // BS, B)
    return pl.pallas_call(
        _add_body,
        grid=grid,
        in_specs=[
            pl.BlockSpec((1, BS, D), lambda i, j: (j, i, 0)),
            pl.BlockSpec((BS, D), lambda i, j: (i, 0)),
        ],
        out_specs=pl.BlockSpec((1, BS, D), lambda i, j: (j, i, 0)),
        out_shape=jax.ShapeDtypeStruct((B, S, D), x.dtype),
    )(x, emb)
